# baseline (speedup 1.0000x reference)
import functools

import jax
import jax.numpy as jnp
from jax import lax
from jax.experimental import pallas as pl
from jax.experimental.pallas import tpu as pltpu
from jax.experimental.pallas import tpu_sc as plsc

N = 10000
E = 320000
D = 128
DE = 144
NC = 2
NS = 16
NW = NC * NS
NP = 10240
CH = 128
NCHG = E // CH
NKB = NCHG // NW
NKR = NCHG % NW
RPT = NP // NS
ZCH = 128
NZ = RPT // ZCH
SW = 16
RB = 1000
NRB = N // RB

_mesh = plsc.VectorSubcoreMesh(
    core_axis_name="c", subcore_axis_name="s", num_cores=NC, num_subcores=NS)


def _zero_fill(buf, rows, width):
  def body(i, _):
    for j in range(width // 16):
      buf[i, pl.ds(j * 16, 16)] = jnp.zeros((16,), jnp.float32)
    return 0
  lax.fori_loop(0, rows, body, 0, unroll=False)


def _chunk_base(wid, k):
  return pl.multiple_of((wid + NW * k) * CH, 8)


@functools.partial(
    pl.kernel,
    out_type=jax.ShapeDtypeStruct((NC, NP, SW), jnp.float32),
    mesh=_mesh,
    scratch_types=[
        pltpu.VMEM_SHARED((NP, SW), jnp.float32),
        pltpu.VMEM((CH,), jnp.int32),
        pltpu.VMEM((CH,), jnp.int32),
        pltpu.VMEM((CH, SW), jnp.float32),
        pltpu.VMEM((ZCH, SW), jnp.float32),
        pltpu.SemaphoreType.DMA,
        pltpu.SemaphoreType.DMA,
    ],
)
def _sc_degree(dst_hbm, deg_out, shared_deg, idx0, idx1, ones_v, zrow_v,
               isem0, isem1):
  cid = lax.axis_index("c")
  sid = lax.axis_index("s")
  wid = sid * NC + cid
  nk = NKB + jnp.where(wid < NKR, 1, 0)

  def fill_ones(i, _):
    ones_v[i, :] = jnp.full((SW,), 1.0, jnp.float32)
    return 0
  lax.fori_loop(0, CH, fill_ones, 0, unroll=False)
  _zero_fill(zrow_v, ZCH, SW)

  for z in range(NZ):
    r0 = pl.multiple_of(sid * RPT + z * ZCH, ZCH)
    pltpu.sync_copy(zrow_v, shared_deg.at[pl.ds(r0, ZCH)])
  plsc.subcore_barrier()

  bufs = ((idx0, isem0), (idx1, isem1))

  def fire_idx(b, k):
    idx, isem = bufs[b]
    pltpu.async_copy(dst_hbm.at[pl.ds(_chunk_base(wid, k), CH)], idx, isem)

  def wait_idx(b):
    idx, isem = bufs[b]
    pltpu.make_async_copy(dst_hbm.at[pl.ds(0, CH)], idx, isem).wait()

  fire_idx(0, 0)

  @pl.when(nk > 1)
  def _():
    fire_idx(1, 1)

  def body(k, _):
    def step(a, b):
      idx_a, _ = bufs[a]
      wait_idx(a)
      pltpu.sync_copy(ones_v, shared_deg.at[idx_a], add=True)

      @pl.when(k + 2 < nk)
      def _():
        fire_idx(a, k + 2)

    @pl.when(k % 2 == 0)
    def _():
      step(0, 1)

    @pl.when(k % 2 == 1)
    def _():
      step(1, 0)
    return 0
  lax.fori_loop(0, nk, body, 0, unroll=False)
  plsc.subcore_barrier()

  for z in range(NZ):
    r0 = pl.multiple_of(sid * RPT + z * ZCH, ZCH)
    pltpu.sync_copy(shared_deg.at[pl.ds(r0, ZCH)], zrow_v)
    pltpu.sync_copy(zrow_v, deg_out.at[cid, pl.ds(r0, ZCH)])


@functools.partial(
    pl.kernel,
    out_type=jax.ShapeDtypeStruct((NC, NP, DE), jnp.float32),
    mesh=_mesh,
    scratch_types=[
        pltpu.VMEM_SHARED((NP, DE), jnp.float32),
        pltpu.VMEM((CH,), jnp.int32),
        pltpu.VMEM((CH,), jnp.int32),
        pltpu.VMEM((CH,), jnp.int32),
        pltpu.VMEM((CH,), jnp.int32),
        pltpu.VMEM((CH, DE), jnp.float32),
        pltpu.VMEM((CH, DE), jnp.float32),
        pltpu.SemaphoreType.DMA,
        pltpu.SemaphoreType.DMA,
        pltpu.SemaphoreType.DMA,
        pltpu.SemaphoreType.DMA,
    ],
    compiler_params=pltpu.CompilerParams(use_tc_tiling_on_sc=False),
)
def _sc_edges(src_hbm, dst_hbm, hext_hbm, gext_out, shared_g,
              idxs0, idxd0, idxs1, idxd1, rows0, rows1,
              isem0, isem1, gsem0, gsem1):
  zbuf_v = rows0
  cid = lax.axis_index("c")
  sid = lax.axis_index("s")
  wid = sid * NC + cid
  nk = NKB + jnp.where(wid < NKR, 1, 0)

  _zero_fill(zbuf_v, ZCH, DE)
  for z in range(NZ):
    r0 = pl.multiple_of(sid * RPT + z * ZCH, ZCH)
    pltpu.sync_copy(zbuf_v, shared_g.at[pl.ds(r0, ZCH)])
  plsc.subcore_barrier()

  bufs = ((idxs0, idxd0, rows0, isem0, gsem0),
          (idxs1, idxd1, rows1, isem1, gsem1))

  def fire_idx(b, k):
    idxs, idxd, _, isem, _ = bufs[b]
    base = _chunk_base(wid, k)
    pltpu.async_copy(src_hbm.at[pl.ds(base, CH)], idxs, isem)
    pltpu.async_copy(dst_hbm.at[pl.ds(base, CH)], idxd, isem)

  def wait_idx(b):
    idxs, idxd, _, isem, _ = bufs[b]
    pltpu.make_async_copy(src_hbm.at[pl.ds(0, CH)], idxs, isem).wait()
    pltpu.make_async_copy(dst_hbm.at[pl.ds(0, CH)], idxd, isem).wait()

  def fire_gather(b):
    idxs, _, rows, _, gsem = bufs[b]
    pltpu.async_copy(hext_hbm.at[idxs], rows, gsem)

  def wait_gather(b):
    idxs, _, rows, _, gsem = bufs[b]
    pltpu.make_async_copy(hext_hbm.at[idxs], rows, gsem).wait()

  fire_idx(0, 0)
  wait_idx(0)
  fire_gather(0)

  @pl.when(nk > 1)
  def _():
    fire_idx(1, 1)

  def body(k, _):
    def step(a, b):
      _, idxd_a, rows_a, _, _ = bufs[a]
      wait_gather(a)

      @pl.when(k + 1 < nk)
      def _():
        wait_idx(b)
        fire_gather(b)

      pltpu.sync_copy(rows_a, shared_g.at[idxd_a], add=True)

      @pl.when(k + 2 < nk)
      def _():
        fire_idx(a, k + 2)

    @pl.when(k % 2 == 0)
    def _():
      step(0, 1)

    @pl.when(k % 2 == 1)
    def _():
      step(1, 0)
    return 0
  lax.fori_loop(0, nk, body, 0, unroll=False)
  plsc.subcore_barrier()

  for z in range(NZ):
    r0 = pl.multiple_of(sid * RPT + z * ZCH, ZCH)
    pltpu.sync_copy(shared_g.at[pl.ds(r0, ZCH)], zbuf_v)
    pltpu.sync_copy(zbuf_v, gext_out.at[cid, pl.ds(r0, ZCH)])


def _tc_prep_body(deg_ref, f_ref, h_ref):
  deg = deg_ref[0, :, 0:1] + deg_ref[1, :, 0:1]
  isq = lax.rsqrt(jnp.maximum(deg, 1.0))
  h = jnp.concatenate(
      [f_ref[...] * isq, isq, jnp.zeros((RB, DE - D - 1), jnp.float32)],
      axis=1)
  h_ref[...] = h


def _tc_prep(deg_p, feature):
  return pl.pallas_call(
      _tc_prep_body,
      grid=(NRB,),
      in_specs=[
          pl.BlockSpec((NC, RB, SW), lambda i: (0, i, 0)),
          pl.BlockSpec((RB, D), lambda i: (i, 0)),
      ],
      out_specs=pl.BlockSpec((RB, DE), lambda i: (i, 0)),
      out_shape=jax.ShapeDtypeStruct((N, DE), jnp.float32),
  )(deg_p, feature)


def _tc_final_body(f_ref, h_ref, g_ref, wg_ref, we_ref, o_ref):
  gext = g_ref[0] + g_ref[1]
  hf = h_ref[:, 0:D]
  s1 = gext[:, D:D + 1]
  g = gext[:, 0:D]
  f = f_ref[...]
  a = f + hf * s1
  b = hf * g
  dn = (((1,), (1,)), ((), ()))
  r = lax.dot_general(a, wg_ref[...], dn,
                      preferred_element_type=jnp.float32,
                      precision=lax.Precision.HIGHEST)
  r = r + lax.dot_general(b, we_ref[...], dn,
                          preferred_element_type=jnp.float32,
                          precision=lax.Precision.HIGHEST)
  r = jnp.where(r >= 0, r, 0.2 * r)
  nrm = jnp.sqrt(jnp.sum(r * r, axis=1, keepdims=True))
  o_ref[...] = r / jnp.maximum(nrm, 1e-12)


def _tc_final(feature, hext, g_p, W_gcn, W_enh):
  return pl.pallas_call(
      _tc_final_body,
      grid=(NRB,),
      in_specs=[
          pl.BlockSpec((RB, D), lambda i: (i, 0)),
          pl.BlockSpec((RB, DE), lambda i: (i, 0)),
          pl.BlockSpec((NC, RB, DE), lambda i: (0, i, 0)),
          pl.BlockSpec((D, D), lambda i: (0, 0)),
          pl.BlockSpec((D, D), lambda i: (0, 0)),
      ],
      out_specs=pl.BlockSpec((RB, D), lambda i: (i, 0)),
      out_shape=jax.ShapeDtypeStruct((N, D), jnp.float32),
  )(feature, hext, g_p, W_gcn, W_enh)


def kernel(feature, edge_index, W_gcn, W_enh):
  ei = edge_index.astype(jnp.int32)
  src = ei[0]
  dst = ei[1]
  deg_p = _sc_degree(dst)
  hext = _tc_prep(deg_p, feature)
  g_p = _sc_edges(src, dst, hext)
  return _tc_final(feature, hext, g_p, W_gcn, W_enh)


# speedup vs baseline: 1.0016x; 1.0016x over previous
import functools

import jax
import jax.numpy as jnp
from jax import lax
from jax.experimental import pallas as pl
from jax.experimental.pallas import tpu as pltpu
from jax.experimental.pallas import tpu_sc as plsc

N = 10000
E = 320000
D = 128
DE = 144
NC = 2
NS = 16
NW = NC * NS
NP = 10240
CH = 128
NCHG = E // CH
NKB = NCHG // NW
NKR = NCHG % NW
RPT = NP // NS
ZCH = 128
NZ = RPT // ZCH
SW = 16
RB = 1000
NRB = N // RB

_mesh = plsc.VectorSubcoreMesh(
    core_axis_name="c", subcore_axis_name="s", num_cores=NC, num_subcores=NS)


def _zero_fill(buf, rows, width):
  def body(i, _):
    for j in range(width // 16):
      buf[i, pl.ds(j * 16, 16)] = jnp.zeros((16,), jnp.float32)
    return 0
  lax.fori_loop(0, rows, body, 0, unroll=False)


def _chunk_base(wid, k):
  return pl.multiple_of((wid + NW * k) * CH, 8)


@functools.partial(
    pl.kernel,
    out_type=jax.ShapeDtypeStruct((NC, NP, SW), jnp.float32),
    mesh=_mesh,
    scratch_types=[
        pltpu.VMEM_SHARED((NP, SW), jnp.float32),
        pltpu.VMEM((CH,), jnp.int32),
        pltpu.VMEM((CH,), jnp.int32),
        pltpu.VMEM((CH, SW), jnp.float32),
        pltpu.VMEM((ZCH, SW), jnp.float32),
        pltpu.SemaphoreType.DMA,
        pltpu.SemaphoreType.DMA,
    ],
    compiler_params=pltpu.CompilerParams(use_tc_tiling_on_sc=False),
)
def _sc_degree(dst_hbm, deg_out, shared_deg, idx0, idx1, ones_v, zrow_v,
               isem0, isem1):
  cid = lax.axis_index("c")
  sid = lax.axis_index("s")
  wid = sid * NC + cid
  nk = NKB + jnp.where(wid < NKR, 1, 0)

  def fill_ones(i, _):
    ones_v[i, :] = jnp.full((SW,), 1.0, jnp.float32)
    return 0
  lax.fori_loop(0, CH, fill_ones, 0, unroll=False)
  _zero_fill(zrow_v, ZCH, SW)

  for z in range(NZ):
    r0 = pl.multiple_of(sid * RPT + z * ZCH, ZCH)
    pltpu.sync_copy(zrow_v, shared_deg.at[pl.ds(r0, ZCH)])
  plsc.subcore_barrier()

  bufs = ((idx0, isem0), (idx1, isem1))

  def fire_idx(b, k):
    idx, isem = bufs[b]
    pltpu.async_copy(dst_hbm.at[pl.ds(_chunk_base(wid, k), CH)], idx, isem)

  def wait_idx(b):
    idx, isem = bufs[b]
    pltpu.make_async_copy(dst_hbm.at[pl.ds(0, CH)], idx, isem).wait()

  fire_idx(0, 0)

  @pl.when(nk > 1)
  def _():
    fire_idx(1, 1)

  def body(k, _):
    def step(a, b):
      idx_a, _ = bufs[a]
      wait_idx(a)
      pltpu.sync_copy(ones_v, shared_deg.at[idx_a], add=True)

      @pl.when(k + 2 < nk)
      def _():
        fire_idx(a, k + 2)

    @pl.when(k % 2 == 0)
    def _():
      step(0, 1)

    @pl.when(k % 2 == 1)
    def _():
      step(1, 0)
    return 0
  lax.fori_loop(0, nk, body, 0, unroll=False)
  plsc.subcore_barrier()

  for z in range(NZ):
    r0 = pl.multiple_of(sid * RPT + z * ZCH, ZCH)
    pltpu.sync_copy(shared_deg.at[pl.ds(r0, ZCH)], zrow_v)
    pltpu.sync_copy(zrow_v, deg_out.at[cid, pl.ds(r0, ZCH)])


@functools.partial(
    pl.kernel,
    out_type=jax.ShapeDtypeStruct((NC, NP, DE), jnp.float32),
    mesh=_mesh,
    scratch_types=[
        pltpu.VMEM_SHARED((NP, DE), jnp.float32),
        pltpu.VMEM((CH,), jnp.int32),
        pltpu.VMEM((CH,), jnp.int32),
        pltpu.VMEM((CH,), jnp.int32),
        pltpu.VMEM((CH,), jnp.int32),
        pltpu.VMEM((CH, DE), jnp.float32),
        pltpu.VMEM((CH, DE), jnp.float32),
        pltpu.SemaphoreType.DMA,
        pltpu.SemaphoreType.DMA,
        pltpu.SemaphoreType.DMA,
        pltpu.SemaphoreType.DMA,
    ],
    compiler_params=pltpu.CompilerParams(use_tc_tiling_on_sc=False),
)
def _sc_edges(src_hbm, dst_hbm, hext_hbm, gext_out, shared_g,
              idxs0, idxd0, idxs1, idxd1, rows0, rows1,
              isem0, isem1, gsem0, gsem1):
  zbuf_v = rows0
  cid = lax.axis_index("c")
  sid = lax.axis_index("s")
  wid = sid * NC + cid
  nk = NKB + jnp.where(wid < NKR, 1, 0)

  _zero_fill(zbuf_v, ZCH, DE)
  for z in range(NZ):
    r0 = pl.multiple_of(sid * RPT + z * ZCH, ZCH)
    pltpu.sync_copy(zbuf_v, shared_g.at[pl.ds(r0, ZCH)])
  plsc.subcore_barrier()

  bufs = ((idxs0, idxd0, rows0, isem0, gsem0),
          (idxs1, idxd1, rows1, isem1, gsem1))

  def fire_idx(b, k):
    idxs, idxd, _, isem, _ = bufs[b]
    base = _chunk_base(wid, k)
    pltpu.async_copy(src_hbm.at[pl.ds(base, CH)], idxs, isem)
    pltpu.async_copy(dst_hbm.at[pl.ds(base, CH)], idxd, isem)

  def wait_idx(b):
    idxs, idxd, _, isem, _ = bufs[b]
    pltpu.make_async_copy(src_hbm.at[pl.ds(0, CH)], idxs, isem).wait()
    pltpu.make_async_copy(dst_hbm.at[pl.ds(0, CH)], idxd, isem).wait()

  def fire_gather(b):
    idxs, _, rows, _, gsem = bufs[b]
    pltpu.async_copy(hext_hbm.at[idxs], rows, gsem)

  def wait_gather(b):
    idxs, _, rows, _, gsem = bufs[b]
    pltpu.make_async_copy(hext_hbm.at[idxs], rows, gsem).wait()

  fire_idx(0, 0)
  wait_idx(0)
  fire_gather(0)

  @pl.when(nk > 1)
  def _():
    fire_idx(1, 1)

  def body(k, _):
    def step(a, b):
      _, idxd_a, rows_a, _, _ = bufs[a]
      wait_gather(a)

      @pl.when(k + 1 < nk)
      def _():
        wait_idx(b)
        fire_gather(b)

      pltpu.sync_copy(rows_a, shared_g.at[idxd_a], add=True)

      @pl.when(k + 2 < nk)
      def _():
        fire_idx(a, k + 2)

    @pl.when(k % 2 == 0)
    def _():
      step(0, 1)

    @pl.when(k % 2 == 1)
    def _():
      step(1, 0)
    return 0
  lax.fori_loop(0, nk, body, 0, unroll=False)
  plsc.subcore_barrier()

  for z in range(NZ):
    r0 = pl.multiple_of(sid * RPT + z * ZCH, ZCH)
    pltpu.sync_copy(shared_g.at[pl.ds(r0, ZCH)], zbuf_v)
    pltpu.sync_copy(zbuf_v, gext_out.at[cid, pl.ds(r0, ZCH)])


def _tc_prep_body(deg_ref, f_ref, h_ref):
  deg = deg_ref[0, :, 0:1] + deg_ref[1, :, 0:1]
  isq = lax.rsqrt(jnp.maximum(deg, 1.0))
  h = jnp.concatenate(
      [f_ref[...] * isq, isq, jnp.zeros((RB, DE - D - 1), jnp.float32)],
      axis=1)
  h_ref[...] = h


def _tc_prep(deg_p, feature):
  return pl.pallas_call(
      _tc_prep_body,
      grid=(NRB,),
      in_specs=[
          pl.BlockSpec((NC, RB, SW), lambda i: (0, i, 0)),
          pl.BlockSpec((RB, D), lambda i: (i, 0)),
      ],
      out_specs=pl.BlockSpec((RB, DE), lambda i: (i, 0)),
      out_shape=jax.ShapeDtypeStruct((N, DE), jnp.float32),
  )(deg_p, feature)


def _tc_final_body(f_ref, h_ref, g_ref, wg_ref, we_ref, o_ref):
  gext = g_ref[0] + g_ref[1]
  hf = h_ref[:, 0:D]
  s1 = gext[:, D:D + 1]
  g = gext[:, 0:D]
  f = f_ref[...]
  a = f + hf * s1
  b = hf * g
  dn = (((1,), (1,)), ((), ()))
  r = lax.dot_general(a, wg_ref[...], dn,
                      preferred_element_type=jnp.float32,
                      precision=lax.Precision.HIGHEST)
  r = r + lax.dot_general(b, we_ref[...], dn,
                          preferred_element_type=jnp.float32,
                          precision=lax.Precision.HIGHEST)
  r = jnp.where(r >= 0, r, 0.2 * r)
  nrm = jnp.sqrt(jnp.sum(r * r, axis=1, keepdims=True))
  o_ref[...] = r / jnp.maximum(nrm, 1e-12)


def _tc_final(feature, hext, g_p, W_gcn, W_enh):
  return pl.pallas_call(
      _tc_final_body,
      grid=(NRB,),
      in_specs=[
          pl.BlockSpec((RB, D), lambda i: (i, 0)),
          pl.BlockSpec((RB, DE), lambda i: (i, 0)),
          pl.BlockSpec((NC, RB, DE), lambda i: (0, i, 0)),
          pl.BlockSpec((D, D), lambda i: (0, 0)),
          pl.BlockSpec((D, D), lambda i: (0, 0)),
      ],
      out_specs=pl.BlockSpec((RB, D), lambda i: (i, 0)),
      out_shape=jax.ShapeDtypeStruct((N, D), jnp.float32),
  )(feature, hext, g_p, W_gcn, W_enh)


def kernel(feature, edge_index, W_gcn, W_enh):
  ei = edge_index.astype(jnp.int32)
  src = ei[0]
  dst = ei[1]
  deg_p = _sc_degree(dst)
  hext = _tc_prep(deg_p, feature)
  g_p = _sc_edges(src, dst, hext)
  return _tc_final(feature, hext, g_p, W_gcn, W_enh)


# speedup vs baseline: 1.0251x; 1.0234x over previous
import functools

import jax
import jax.numpy as jnp
from jax import lax
from jax.experimental import pallas as pl
from jax.experimental.pallas import tpu as pltpu
from jax.experimental.pallas import tpu_sc as plsc

N = 10000
E = 320000
D = 128
DE = 144
NC = 2
NS = 16
NW = NC * NS
NP = 10240
CH = 128
NCHG = E // CH
NKB = NCHG // NW
NKR = NCHG % NW
RPT = NP // NS
ZCH = 128
NZ = RPT // ZCH
SW = 16
RB = 1000
NRB = N // RB

_mesh = plsc.VectorSubcoreMesh(
    core_axis_name="c", subcore_axis_name="s", num_cores=NC, num_subcores=NS)


def _zero_fill(buf, rows, width):
  def body(i, _):
    for j in range(width // 16):
      buf[i, pl.ds(j * 16, 16)] = jnp.zeros((16,), jnp.float32)
    return 0
  lax.fori_loop(0, rows, body, 0, unroll=False)


def _chunk_base(wid, k):
  return pl.multiple_of((wid + NW * k) * CH, 8)


@functools.partial(
    pl.kernel,
    out_type=jax.ShapeDtypeStruct((NC, NP, SW), jnp.float32),
    mesh=_mesh,
    scratch_types=[
        pltpu.VMEM_SHARED((NP, SW), jnp.float32),
        pltpu.VMEM((CH,), jnp.int32),
        pltpu.VMEM((CH,), jnp.int32),
        pltpu.VMEM((CH, SW), jnp.float32),
        pltpu.VMEM((ZCH, SW), jnp.float32),
        pltpu.SemaphoreType.DMA,
        pltpu.SemaphoreType.DMA,
    ],
    compiler_params=pltpu.CompilerParams(use_tc_tiling_on_sc=False),
)
def _sc_degree(dst_hbm, deg_out, shared_deg, idx0, idx1, ones_v, zrow_v,
               isem0, isem1):
  cid = lax.axis_index("c")
  sid = lax.axis_index("s")
  wid = sid * NC + cid
  nk = NKB + jnp.where(wid < NKR, 1, 0)

  def fill_ones(i, _):
    ones_v[i, :] = jnp.full((SW,), 1.0, jnp.float32)
    return 0
  lax.fori_loop(0, CH, fill_ones, 0, unroll=False)
  _zero_fill(zrow_v, ZCH, SW)

  for z in range(NZ):
    r0 = pl.multiple_of(sid * RPT + z * ZCH, ZCH)
    pltpu.sync_copy(zrow_v, shared_deg.at[pl.ds(r0, ZCH)])
  plsc.subcore_barrier()

  bufs = ((idx0, isem0), (idx1, isem1))

  def fire_idx(b, k):
    idx, isem = bufs[b]
    pltpu.async_copy(dst_hbm.at[pl.ds(_chunk_base(wid, k), CH)], idx, isem)

  def wait_idx(b):
    idx, isem = bufs[b]
    pltpu.make_async_copy(dst_hbm.at[pl.ds(0, CH)], idx, isem).wait()

  fire_idx(0, 0)

  @pl.when(nk > 1)
  def _():
    fire_idx(1, 1)

  def body(k, _):
    def step(a, b):
      idx_a, _ = bufs[a]
      wait_idx(a)
      pltpu.sync_copy(ones_v, shared_deg.at[idx_a], add=True)

      @pl.when(k + 2 < nk)
      def _():
        fire_idx(a, k + 2)

    @pl.when(k % 2 == 0)
    def _():
      step(0, 1)

    @pl.when(k % 2 == 1)
    def _():
      step(1, 0)
    return 0
  lax.fori_loop(0, nk, body, 0, unroll=False)
  plsc.subcore_barrier()

  for z in range(NZ):
    r0 = pl.multiple_of(sid * RPT + z * ZCH, ZCH)
    pltpu.sync_copy(shared_deg.at[pl.ds(r0, ZCH)], zrow_v)
    pltpu.sync_copy(zrow_v, deg_out.at[cid, pl.ds(r0, ZCH)])


@functools.partial(
    pl.kernel,
    out_type=(
        jax.ShapeDtypeStruct((NC, NP, D), jnp.float32),
        jax.ShapeDtypeStruct((NC, NP, SW), jnp.float32),
    ),
    mesh=_mesh,
    scratch_types=[
        pltpu.VMEM_SHARED((NP, DE), jnp.float32),
        pltpu.VMEM((CH,), jnp.int32),
        pltpu.VMEM((CH,), jnp.int32),
        pltpu.VMEM((CH,), jnp.int32),
        pltpu.VMEM((CH,), jnp.int32),
        pltpu.VMEM((CH, DE), jnp.float32),
        pltpu.VMEM((CH, DE), jnp.float32),
        pltpu.SemaphoreType.DMA,
        pltpu.SemaphoreType.DMA,
        pltpu.SemaphoreType.DMA,
        pltpu.SemaphoreType.DMA,
    ],
    compiler_params=pltpu.CompilerParams(use_tc_tiling_on_sc=False),
)
def _sc_edges(src_hbm, dst_hbm, hext_hbm, gc_out, s1_out, shared_g,
              idxs0, idxd0, idxs1, idxd1, rows0, rows1,
              isem0, isem1, gsem0, gsem1):
  zbuf_v = rows0
  cid = lax.axis_index("c")
  sid = lax.axis_index("s")
  wid = sid * NC + cid
  nk = NKB + jnp.where(wid < NKR, 1, 0)

  _zero_fill(zbuf_v, ZCH, DE)
  for z in range(NZ):
    r0 = pl.multiple_of(sid * RPT + z * ZCH, ZCH)
    pltpu.sync_copy(zbuf_v, shared_g.at[pl.ds(r0, ZCH)])
  plsc.subcore_barrier()

  bufs = ((idxs0, idxd0, rows0, isem0, gsem0),
          (idxs1, idxd1, rows1, isem1, gsem1))

  def fire_idx(b, k):
    idxs, idxd, _, isem, _ = bufs[b]
    base = _chunk_base(wid, k)
    pltpu.async_copy(src_hbm.at[pl.ds(base, CH)], idxs, isem)
    pltpu.async_copy(dst_hbm.at[pl.ds(base, CH)], idxd, isem)

  def wait_idx(b):
    idxs, idxd, _, isem, _ = bufs[b]
    pltpu.make_async_copy(src_hbm.at[pl.ds(0, CH)], idxs, isem).wait()
    pltpu.make_async_copy(dst_hbm.at[pl.ds(0, CH)], idxd, isem).wait()

  def fire_gather(b):
    idxs, _, rows, _, gsem = bufs[b]
    pltpu.async_copy(hext_hbm.at[idxs], rows, gsem)

  def wait_gather(b):
    idxs, _, rows, _, gsem = bufs[b]
    pltpu.make_async_copy(hext_hbm.at[idxs], rows, gsem).wait()

  fire_idx(0, 0)
  wait_idx(0)
  fire_gather(0)

  @pl.when(nk > 1)
  def _():
    fire_idx(1, 1)

  def body(k, _):
    def step(a, b):
      _, idxd_a, rows_a, _, _ = bufs[a]
      wait_gather(a)

      @pl.when(k + 1 < nk)
      def _():
        wait_idx(b)
        fire_gather(b)

      pltpu.sync_copy(rows_a, shared_g.at[idxd_a], add=True)

      @pl.when(k + 2 < nk)
      def _():
        fire_idx(a, k + 2)

    @pl.when(k % 2 == 0)
    def _():
      step(0, 1)

    @pl.when(k % 2 == 1)
    def _():
      step(1, 0)
    return 0
  lax.fori_loop(0, nk, body, 0, unroll=False)
  plsc.subcore_barrier()

  for z in range(NZ):
    r0 = pl.multiple_of(sid * RPT + z * ZCH, ZCH)
    pltpu.sync_copy(shared_g.at[pl.ds(r0, ZCH)], zbuf_v)

    def splat_s1(r, _):
      v = zbuf_v[r, pl.ds(D, SW)]
      zbuf_v[r, pl.ds(D, SW)] = jnp.full((SW,), v[0], jnp.float32)
      return 0
    lax.fori_loop(0, ZCH, splat_s1, 0, unroll=False)
    pltpu.sync_copy(zbuf_v.at[:, pl.ds(0, D)], gc_out.at[cid, pl.ds(r0, ZCH)])
    pltpu.sync_copy(zbuf_v.at[:, pl.ds(D, SW)], s1_out.at[cid, pl.ds(r0, ZCH)])


def _tc_prep_body(deg_ref, f_ref, h_ref, isq16_ref):
  deg = deg_ref[0, :, 0:1] + deg_ref[1, :, 0:1]
  isq = lax.rsqrt(jnp.maximum(deg, 1.0))
  h = jnp.concatenate(
      [f_ref[...] * isq, isq, jnp.zeros((RB, DE - D - 1), jnp.float32)],
      axis=1)
  h_ref[...] = h
  isq16_ref[...] = jnp.broadcast_to(isq, (RB, SW))


def _tc_prep(deg_p, feature):
  return pl.pallas_call(
      _tc_prep_body,
      grid=(NRB,),
      in_specs=[
          pl.BlockSpec((NC, RB, SW), lambda i: (0, i, 0)),
          pl.BlockSpec((RB, D), lambda i: (i, 0)),
      ],
      out_specs=[
          pl.BlockSpec((RB, DE), lambda i: (i, 0)),
          pl.BlockSpec((RB, SW), lambda i: (i, 0)),
      ],
      out_shape=[
          jax.ShapeDtypeStruct((N, DE), jnp.float32),
          jax.ShapeDtypeStruct((N, SW), jnp.float32),
      ],
  )(deg_p, feature)


def _tc_final_body(f_ref, isq16_ref, g_ref, s1_ref, wg_ref, we_ref, o_ref):
  g = g_ref[0] + g_ref[1]
  s1 = s1_ref[0, :, 0:1] + s1_ref[1, :, 0:1]
  f = f_ref[...]
  hf = f * isq16_ref[:, 0:1]
  a = f + hf * s1
  b = hf * g
  dn = (((1,), (1,)), ((), ()))
  r = lax.dot_general(a, wg_ref[...], dn,
                      preferred_element_type=jnp.float32,
                      precision=lax.Precision.HIGHEST)
  r = r + lax.dot_general(b, we_ref[...], dn,
                          preferred_element_type=jnp.float32,
                          precision=lax.Precision.HIGHEST)
  r = jnp.where(r >= 0, r, 0.2 * r)
  nrm = jnp.sqrt(jnp.sum(r * r, axis=1, keepdims=True))
  o_ref[...] = r / jnp.maximum(nrm, 1e-12)


def _tc_final(feature, isq16, g_p, s1_p, W_gcn, W_enh):
  return pl.pallas_call(
      _tc_final_body,
      grid=(NRB,),
      in_specs=[
          pl.BlockSpec((RB, D), lambda i: (i, 0)),
          pl.BlockSpec((RB, SW), lambda i: (i, 0)),
          pl.BlockSpec((NC, RB, D), lambda i: (0, i, 0)),
          pl.BlockSpec((NC, RB, SW), lambda i: (0, i, 0)),
          pl.BlockSpec((D, D), lambda i: (0, 0)),
          pl.BlockSpec((D, D), lambda i: (0, 0)),
      ],
      out_specs=pl.BlockSpec((RB, D), lambda i: (i, 0)),
      out_shape=jax.ShapeDtypeStruct((N, D), jnp.float32),
  )(feature, isq16, g_p, s1_p, W_gcn, W_enh)


def kernel(feature, edge_index, W_gcn, W_enh):
  ei = edge_index.astype(jnp.int32)
  src = ei[0]
  dst = ei[1]
  deg_p = _sc_degree(dst)
  hext, isq16 = _tc_prep(deg_p, feature)
  g_p, s1_p = _sc_edges(src, dst, hext)
  return _tc_final(feature, isq16, g_p, s1_p, W_gcn, W_enh)


# speedup vs baseline: 1.0259x; 1.0008x over previous
import functools

import jax
import jax.numpy as jnp
from jax import lax
from jax.experimental import pallas as pl
from jax.experimental.pallas import tpu as pltpu
from jax.experimental.pallas import tpu_sc as plsc

N = 10000
E = 320000
D = 128
DE = 144
NC = 2
NS = 16
NW = NC * NS
NP = 10240
CH = 128
NCHG = E // CH
NKB = NCHG // NW
NKR = NCHG % NW
RPT = NP // NS
ZCH = 128
NZ = RPT // ZCH
SW = 16
RB = 1000
NRB = N // RB

_mesh = plsc.VectorSubcoreMesh(
    core_axis_name="c", subcore_axis_name="s", num_cores=NC, num_subcores=NS)


def _zero_fill(buf, rows, width):
  def body(i, _):
    for j in range(width // 16):
      buf[i, pl.ds(j * 16, 16)] = jnp.zeros((16,), jnp.float32)
    return 0
  lax.fori_loop(0, rows, body, 0, unroll=False)


def _chunk_base(wid, k):
  return pl.multiple_of((wid + NW * k) * CH, 8)


@functools.partial(
    pl.kernel,
    out_type=jax.ShapeDtypeStruct((NC, NP, SW), jnp.float32),
    mesh=_mesh,
    scratch_types=[
        pltpu.VMEM_SHARED((NP, SW), jnp.float32),
        pltpu.VMEM((CH,), jnp.int32),
        pltpu.VMEM((CH,), jnp.int32),
        pltpu.VMEM((CH, SW), jnp.float32),
        pltpu.VMEM((ZCH, SW), jnp.float32),
        pltpu.SemaphoreType.DMA,
        pltpu.SemaphoreType.DMA,
    ],
    compiler_params=pltpu.CompilerParams(use_tc_tiling_on_sc=False),
)
def _sc_degree(dst_hbm, deg_out, shared_deg, idx0, idx1, ones_v, zrow_v,
               isem0, isem1):
  cid = lax.axis_index("c")
  sid = lax.axis_index("s")
  wid = sid * NC + cid
  nk = NKB + jnp.where(wid < NKR, 1, 0)

  def fill_ones(i, _):
    ones_v[i, :] = jnp.full((SW,), 1.0, jnp.float32)
    return 0
  lax.fori_loop(0, CH, fill_ones, 0, unroll=False)
  _zero_fill(zrow_v, ZCH, SW)

  for z in range(NZ):
    r0 = pl.multiple_of(sid * RPT + z * ZCH, ZCH)
    pltpu.sync_copy(zrow_v, shared_deg.at[pl.ds(r0, ZCH)])
  plsc.subcore_barrier()

  bufs = ((idx0, isem0), (idx1, isem1))

  def fire_idx(b, k):
    idx, isem = bufs[b]
    pltpu.async_copy(dst_hbm.at[pl.ds(_chunk_base(wid, k), CH)], idx, isem)

  def wait_idx(b):
    idx, isem = bufs[b]
    pltpu.make_async_copy(dst_hbm.at[pl.ds(0, CH)], idx, isem).wait()

  fire_idx(0, 0)

  @pl.when(nk > 1)
  def _():
    fire_idx(1, 1)

  def body(k, _):
    def step(a, b):
      idx_a, _ = bufs[a]
      wait_idx(a)
      pltpu.sync_copy(ones_v, shared_deg.at[idx_a], add=True)

      @pl.when(k + 2 < nk)
      def _():
        fire_idx(a, k + 2)

    @pl.when(k % 2 == 0)
    def _():
      step(0, 1)

    @pl.when(k % 2 == 1)
    def _():
      step(1, 0)
    return 0
  lax.fori_loop(0, nk, body, 0, unroll=False)
  plsc.subcore_barrier()

  for z in range(NZ):
    r0 = pl.multiple_of(sid * RPT + z * ZCH, ZCH)
    pltpu.sync_copy(shared_deg.at[pl.ds(r0, ZCH)], zrow_v)
    pltpu.sync_copy(zrow_v, deg_out.at[cid, pl.ds(r0, ZCH)])


@functools.partial(
    pl.kernel,
    out_type=(
        jax.ShapeDtypeStruct((NC, NP, D), jnp.float32),
        jax.ShapeDtypeStruct((NC, NP, SW), jnp.float32),
    ),
    mesh=_mesh,
    scratch_types=[
        pltpu.VMEM_SHARED((NP, DE), jnp.float32),
        pltpu.VMEM((CH,), jnp.int32),
        pltpu.VMEM((CH,), jnp.int32),
        pltpu.VMEM((CH,), jnp.int32),
        pltpu.VMEM((CH,), jnp.int32),
        pltpu.VMEM((CH,), jnp.int32),
        pltpu.VMEM((CH,), jnp.int32),
        pltpu.VMEM((CH,), jnp.int32),
        pltpu.VMEM((CH,), jnp.int32),
        pltpu.VMEM((CH, DE), jnp.float32),
        pltpu.VMEM((CH, DE), jnp.float32),
        pltpu.SemaphoreType.DMA,
        pltpu.SemaphoreType.DMA,
        pltpu.SemaphoreType.DMA,
        pltpu.SemaphoreType.DMA,
        pltpu.SemaphoreType.DMA,
        pltpu.SemaphoreType.DMA,
        pltpu.SemaphoreType.DMA,
        pltpu.SemaphoreType.DMA,
    ],
    compiler_params=pltpu.CompilerParams(use_tc_tiling_on_sc=False),
)
def _sc_edges(src_hbm, dst_hbm, hext_hbm, gc_out, s1_out, shared_g,
              is0, id0, is1, id1, is2, id2, is3, id3, rows0, rows1,
              im0, im1, im2, im3, gsem0, gsem1, ssem0, ssem1):
  zbuf_v = rows0
  cid = lax.axis_index("c")
  sid = lax.axis_index("s")
  wid = sid * NC + cid
  nk = NKB + jnp.where(wid < NKR, 1, 0)

  _zero_fill(zbuf_v, ZCH, DE)
  for z in range(NZ):
    r0 = pl.multiple_of(sid * RPT + z * ZCH, ZCH)
    pltpu.sync_copy(zbuf_v, shared_g.at[pl.ds(r0, ZCH)])
  plsc.subcore_barrier()

  ibufs = ((is0, id0, im0), (is1, id1, im1), (is2, id2, im2), (is3, id3, im3))
  rbufs = ((rows0, gsem0, ssem0), (rows1, gsem1, ssem1))

  def fire_idx(i4, k):
    idxs, idxd, isem = ibufs[i4]
    base = _chunk_base(wid, k)
    pltpu.async_copy(src_hbm.at[pl.ds(base, CH)], idxs, isem)
    pltpu.async_copy(dst_hbm.at[pl.ds(base, CH)], idxd, isem)

  def wait_idx(i4):
    idxs, idxd, isem = ibufs[i4]
    pltpu.make_async_copy(src_hbm.at[pl.ds(0, CH)], idxs, isem).wait()
    pltpu.make_async_copy(dst_hbm.at[pl.ds(0, CH)], idxd, isem).wait()

  def fire_gather(r2, i4):
    rows, gsem, _ = rbufs[r2]
    pltpu.async_copy(hext_hbm.at[ibufs[i4][0]], rows, gsem)

  def wait_gather(r2, i4):
    rows, gsem, _ = rbufs[r2]
    pltpu.make_async_copy(hext_hbm.at[ibufs[i4][0]], rows, gsem).wait()

  def fire_scatter(r2, i4):
    rows, _, ssem = rbufs[r2]
    pltpu.async_copy(rows, shared_g.at[ibufs[i4][1]], ssem, add=True)

  def wait_scatter(r2, i4):
    rows, _, ssem = rbufs[r2]
    pltpu.make_async_copy(rows, shared_g.at[ibufs[i4][1]], ssem).wait()

  fire_idx(0, 0)

  @pl.when(nk > 1)
  def _():
    fire_idx(1, 1)

  @pl.when(nk > 2)
  def _():
    fire_idx(2, 2)

  wait_idx(0)
  fire_gather(0, 0)

  def body(k, _):
    def step(m4):
      b = m4 % 2
      nb = 1 - b
      i_n = (m4 + 1) % 4
      i_p = (m4 + 3) % 4
      wait_gather(b, m4)

      @pl.when(k >= 1)
      def _():
        wait_scatter(nb, i_p)

      fire_scatter(b, m4)

      @pl.when(k + 1 < nk)
      def _():
        wait_idx(i_n)
        fire_gather(nb, i_n)

        @pl.when(k + 3 < nk)
        def _():
          fire_idx(i_p, k + 3)

    for m in range(4):
      @pl.when(k % 4 == m)
      def _(m=m):
        step(m)
    return 0
  lax.fori_loop(0, nk, body, 0, unroll=False)

  for m in range(4):
    @pl.when((nk - 1) % 4 == m)
    def _(m=m):
      wait_scatter(m % 2, m)
  plsc.subcore_barrier()

  for z in range(NZ):
    r0 = pl.multiple_of(sid * RPT + z * ZCH, ZCH)
    pltpu.sync_copy(shared_g.at[pl.ds(r0, ZCH)], zbuf_v)

    def splat_s1(r, _):
      v = zbuf_v[r, pl.ds(D, SW)]
      zbuf_v[r, pl.ds(D, SW)] = jnp.full((SW,), v[0], jnp.float32)
      return 0
    lax.fori_loop(0, ZCH, splat_s1, 0, unroll=False)
    pltpu.sync_copy(zbuf_v.at[:, pl.ds(0, D)], gc_out.at[cid, pl.ds(r0, ZCH)])
    pltpu.sync_copy(zbuf_v.at[:, pl.ds(D, SW)], s1_out.at[cid, pl.ds(r0, ZCH)])


def _tc_prep_body(deg_ref, f_ref, h_ref, isq16_ref):
  deg = deg_ref[0, :, 0:1] + deg_ref[1, :, 0:1]
  isq = lax.rsqrt(jnp.maximum(deg, 1.0))
  h = jnp.concatenate(
      [f_ref[...] * isq, isq, jnp.zeros((RB, DE - D - 1), jnp.float32)],
      axis=1)
  h_ref[...] = h
  isq16_ref[...] = jnp.broadcast_to(isq, (RB, SW))


def _tc_prep(deg_p, feature):
  return pl.pallas_call(
      _tc_prep_body,
      grid=(NRB,),
      in_specs=[
          pl.BlockSpec((NC, RB, SW), lambda i: (0, i, 0)),
          pl.BlockSpec((RB, D), lambda i: (i, 0)),
      ],
      out_specs=[
          pl.BlockSpec((RB, DE), lambda i: (i, 0)),
          pl.BlockSpec((RB, SW), lambda i: (i, 0)),
      ],
      out_shape=[
          jax.ShapeDtypeStruct((N, DE), jnp.float32),
          jax.ShapeDtypeStruct((N, SW), jnp.float32),
      ],
  )(deg_p, feature)


def _tc_final_body(f_ref, isq16_ref, g_ref, s1_ref, wg_ref, we_ref, o_ref):
  g = g_ref[0] + g_ref[1]
  s1 = s1_ref[0, :, 0:1] + s1_ref[1, :, 0:1]
  f = f_ref[...]
  hf = f * isq16_ref[:, 0:1]
  a = f + hf * s1
  b = hf * g
  dn = (((1,), (1,)), ((), ()))
  r = lax.dot_general(a, wg_ref[...], dn,
                      preferred_element_type=jnp.float32,
                      precision=lax.Precision.HIGHEST)
  r = r + lax.dot_general(b, we_ref[...], dn,
                          preferred_element_type=jnp.float32,
                          precision=lax.Precision.HIGHEST)
  r = jnp.where(r >= 0, r, 0.2 * r)
  nrm = jnp.sqrt(jnp.sum(r * r, axis=1, keepdims=True))
  o_ref[...] = r / jnp.maximum(nrm, 1e-12)


def _tc_final(feature, isq16, g_p, s1_p, W_gcn, W_enh):
  return pl.pallas_call(
      _tc_final_body,
      grid=(NRB,),
      in_specs=[
          pl.BlockSpec((RB, D), lambda i: (i, 0)),
          pl.BlockSpec((RB, SW), lambda i: (i, 0)),
          pl.BlockSpec((NC, RB, D), lambda i: (0, i, 0)),
          pl.BlockSpec((NC, RB, SW), lambda i: (0, i, 0)),
          pl.BlockSpec((D, D), lambda i: (0, 0)),
          pl.BlockSpec((D, D), lambda i: (0, 0)),
      ],
      out_specs=pl.BlockSpec((RB, D), lambda i: (i, 0)),
      out_shape=jax.ShapeDtypeStruct((N, D), jnp.float32),
  )(feature, isq16, g_p, s1_p, W_gcn, W_enh)


def kernel(feature, edge_index, W_gcn, W_enh):
  ei = edge_index.astype(jnp.int32)
  src = ei[0]
  dst = ei[1]
  deg_p = _sc_degree(dst)
  hext, isq16 = _tc_prep(deg_p, feature)
  g_p, s1_p = _sc_edges(src, dst, hext)
  return _tc_final(feature, isq16, g_p, s1_p, W_gcn, W_enh)


# speedup vs baseline: 1.0462x; 1.0198x over previous
import functools

import jax
import jax.numpy as jnp
from jax import lax
from jax.experimental import pallas as pl
from jax.experimental.pallas import tpu as pltpu
from jax.experimental.pallas import tpu_sc as plsc

N = 10000
E = 320000
D = 128
DE = 144
NC = 2
NS = 16
NW = NC * NS
NP = 10240
CH = 128
NCHG = E // CH
NKB = NCHG // NW
NKR = NCHG % NW
RPT = NP // NS
ZCH = 128
NZ = RPT // ZCH
SW = 16
RB = 1000
PR = RB * SW // 128
NRB = N // RB

_mesh = plsc.VectorSubcoreMesh(
    core_axis_name="c", subcore_axis_name="s", num_cores=NC, num_subcores=NS)


def _zero_fill(buf, rows, width):
  def body(i, _):
    for j in range(width // 16):
      buf[i, pl.ds(j * 16, 16)] = jnp.zeros((16,), jnp.float32)
    return 0
  lax.fori_loop(0, rows, body, 0, unroll=False)


def _chunk_base(wid, k):
  return pl.multiple_of((wid + NW * k) * CH, 8)


@functools.partial(
    pl.kernel,
    out_type=jax.ShapeDtypeStruct((NC, NP, SW), jnp.float32),
    mesh=_mesh,
    scratch_types=[
        pltpu.VMEM_SHARED((NP, SW), jnp.float32),
        pltpu.VMEM((CH,), jnp.int32),
        pltpu.VMEM((CH,), jnp.int32),
        pltpu.VMEM((CH,), jnp.int32),
        pltpu.VMEM((CH,), jnp.int32),
        pltpu.VMEM((CH, SW), jnp.float32),
        pltpu.VMEM((ZCH, SW), jnp.float32),
        pltpu.SemaphoreType.DMA,
        pltpu.SemaphoreType.DMA,
        pltpu.SemaphoreType.DMA,
        pltpu.SemaphoreType.DMA,
        pltpu.SemaphoreType.DMA,
        pltpu.SemaphoreType.DMA,
    ],
    compiler_params=pltpu.CompilerParams(use_tc_tiling_on_sc=False),
)
def _sc_degree(dst_hbm, deg_out, shared_deg, idx0, idx1, idx2, idx3,
               ones_v, zrow_v, isem0, isem1, isem2, isem3, ssem0, ssem1):
  cid = lax.axis_index("c")
  sid = lax.axis_index("s")
  wid = sid * NC + cid
  nk = NKB + jnp.where(wid < NKR, 1, 0)

  def fill_ones(i, _):
    ones_v[i, :] = jnp.full((SW,), 1.0, jnp.float32)
    return 0
  lax.fori_loop(0, CH, fill_ones, 0, unroll=False)
  _zero_fill(zrow_v, ZCH, SW)

  for z in range(NZ):
    r0 = pl.multiple_of(sid * RPT + z * ZCH, ZCH)
    pltpu.sync_copy(zrow_v, shared_deg.at[pl.ds(r0, ZCH)])
  plsc.subcore_barrier()

  ibufs = ((idx0, isem0), (idx1, isem1), (idx2, isem2), (idx3, isem3))
  ssems = (ssem0, ssem1)

  def fire_idx(i4, k):
    idx, isem = ibufs[i4]
    pltpu.async_copy(dst_hbm.at[pl.ds(_chunk_base(wid, k), CH)], idx, isem)

  def wait_idx(i4):
    idx, isem = ibufs[i4]
    pltpu.make_async_copy(dst_hbm.at[pl.ds(0, CH)], idx, isem).wait()

  def fire_scatter(i4):
    pltpu.async_copy(ones_v, shared_deg.at[ibufs[i4][0]], ssems[i4 % 2],
                     add=True)

  def wait_scatter(i4):
    pltpu.make_async_copy(ones_v, shared_deg.at[ibufs[i4][0]],
                          ssems[i4 % 2]).wait()

  fire_idx(0, 0)

  @pl.when(nk > 1)
  def _():
    fire_idx(1, 1)

  def body(k, _):
    def step(m4):
      i_p2 = (m4 + 2) % 4
      wait_idx(m4)

      @pl.when(k >= 2)
      def _():
        wait_scatter(i_p2)

      fire_scatter(m4)

      @pl.when(k + 2 < nk)
      def _():
        fire_idx(i_p2, k + 2)

    for m in range(4):
      @pl.when(k % 4 == m)
      def _(m=m):
        step(m)
    return 0
  lax.fori_loop(0, nk, body, 0, unroll=False)

  for m in range(4):
    @pl.when((nk - 1) % 4 == m)
    def _(m=m):
      wait_scatter(m)

    @pl.when(((nk - 2) % 4 == m) & (nk >= 2))
    def _(m=m):
      wait_scatter(m)
  plsc.subcore_barrier()

  for z in range(NZ):
    r0 = pl.multiple_of(sid * RPT + z * ZCH, ZCH)
    pltpu.sync_copy(shared_deg.at[pl.ds(r0, ZCH)], zrow_v)
    pltpu.sync_copy(zrow_v, deg_out.at[cid, pl.ds(r0, ZCH)])


@functools.partial(
    pl.kernel,
    out_type=(
        jax.ShapeDtypeStruct((NC, NP, D), jnp.float32),
        jax.ShapeDtypeStruct((NC, NP, SW), jnp.float32),
    ),
    mesh=_mesh,
    scratch_types=[
        pltpu.VMEM_SHARED((NP, DE), jnp.float32),
        pltpu.VMEM((CH,), jnp.int32),
        pltpu.VMEM((CH,), jnp.int32),
        pltpu.VMEM((CH,), jnp.int32),
        pltpu.VMEM((CH,), jnp.int32),
        pltpu.VMEM((CH,), jnp.int32),
        pltpu.VMEM((CH,), jnp.int32),
        pltpu.VMEM((CH,), jnp.int32),
        pltpu.VMEM((CH,), jnp.int32),
        pltpu.VMEM((CH, DE), jnp.float32),
        pltpu.VMEM((CH, DE), jnp.float32),
        pltpu.SemaphoreType.DMA,
        pltpu.SemaphoreType.DMA,
        pltpu.SemaphoreType.DMA,
        pltpu.SemaphoreType.DMA,
        pltpu.SemaphoreType.DMA,
        pltpu.SemaphoreType.DMA,
        pltpu.SemaphoreType.DMA,
        pltpu.SemaphoreType.DMA,
    ],
    compiler_params=pltpu.CompilerParams(use_tc_tiling_on_sc=False),
)
def _sc_edges(src_hbm, dst_hbm, hext_hbm, gc_out, s1_out, shared_g,
              is0, id0, is1, id1, is2, id2, is3, id3, rows0, rows1,
              im0, im1, im2, im3, gsem0, gsem1, ssem0, ssem1):
  zbuf_v = rows0
  cid = lax.axis_index("c")
  sid = lax.axis_index("s")
  wid = sid * NC + cid
  nk = NKB + jnp.where(wid < NKR, 1, 0)

  _zero_fill(zbuf_v, ZCH, DE)
  for z in range(NZ):
    r0 = pl.multiple_of(sid * RPT + z * ZCH, ZCH)
    pltpu.sync_copy(zbuf_v, shared_g.at[pl.ds(r0, ZCH)])
  plsc.subcore_barrier()

  ibufs = ((is0, id0, im0), (is1, id1, im1), (is2, id2, im2), (is3, id3, im3))
  rbufs = ((rows0, gsem0, ssem0), (rows1, gsem1, ssem1))

  def fire_idx(i4, k):
    idxs, idxd, isem = ibufs[i4]
    base = _chunk_base(wid, k)
    pltpu.async_copy(src_hbm.at[pl.ds(base, CH)], idxs, isem)
    pltpu.async_copy(dst_hbm.at[pl.ds(base, CH)], idxd, isem)

  def wait_idx(i4):
    idxs, idxd, isem = ibufs[i4]
    pltpu.make_async_copy(src_hbm.at[pl.ds(0, CH)], idxs, isem).wait()
    pltpu.make_async_copy(dst_hbm.at[pl.ds(0, CH)], idxd, isem).wait()

  def fire_gather(r2, i4):
    rows, gsem, _ = rbufs[r2]
    pltpu.async_copy(hext_hbm.at[ibufs[i4][0]], rows, gsem)

  def wait_gather(r2, i4):
    rows, gsem, _ = rbufs[r2]
    pltpu.make_async_copy(hext_hbm.at[ibufs[i4][0]], rows, gsem).wait()

  def fire_scatter(r2, i4):
    rows, _, ssem = rbufs[r2]
    pltpu.async_copy(rows, shared_g.at[ibufs[i4][1]], ssem, add=True)

  def wait_scatter(r2, i4):
    rows, _, ssem = rbufs[r2]
    pltpu.make_async_copy(rows, shared_g.at[ibufs[i4][1]], ssem).wait()

  fire_idx(0, 0)

  @pl.when(nk > 1)
  def _():
    fire_idx(1, 1)

  @pl.when(nk > 2)
  def _():
    fire_idx(2, 2)

  wait_idx(0)
  fire_gather(0, 0)

  def body(k, _):
    def step(m4):
      b = m4 % 2
      nb = 1 - b
      i_n = (m4 + 1) % 4
      i_p = (m4 + 3) % 4
      wait_gather(b, m4)

      @pl.when(k >= 1)
      def _():
        wait_scatter(nb, i_p)

      fire_scatter(b, m4)

      @pl.when(k + 1 < nk)
      def _():
        wait_idx(i_n)
        fire_gather(nb, i_n)

        @pl.when(k + 3 < nk)
        def _():
          fire_idx(i_p, k + 3)

    for m in range(4):
      @pl.when(k % 4 == m)
      def _(m=m):
        step(m)
    return 0
  lax.fori_loop(0, nk, body, 0, unroll=False)

  for m in range(4):
    @pl.when((nk - 1) % 4 == m)
    def _(m=m):
      wait_scatter(m % 2, m)
  plsc.subcore_barrier()

  for z in range(NZ):
    r0 = pl.multiple_of(sid * RPT + z * ZCH, ZCH)
    pltpu.sync_copy(shared_g.at[pl.ds(r0, ZCH)], zbuf_v)

    def splat_s1(r, _):
      v = zbuf_v[r, pl.ds(D, SW)]
      zbuf_v[r, pl.ds(D, SW)] = jnp.full((SW,), v[0], jnp.float32)
      return 0
    lax.fori_loop(0, ZCH, splat_s1, 0, unroll=False)
    pltpu.sync_copy(zbuf_v.at[:, pl.ds(0, D)], gc_out.at[cid, pl.ds(r0, ZCH)])
    pltpu.sync_copy(zbuf_v.at[:, pl.ds(D, SW)], s1_out.at[cid, pl.ds(r0, ZCH)])


def _tc_prep_body(deg_ref, f_ref, h_ref, isq16_ref):
  deg = deg_ref[0, :, 0:1] + deg_ref[1, :, 0:1]
  isq = lax.rsqrt(jnp.maximum(deg, 1.0))
  h = jnp.concatenate(
      [f_ref[...] * isq, isq, jnp.zeros((RB, DE - D - 1), jnp.float32)],
      axis=1)
  h_ref[...] = h
  isq16_ref[...] = jnp.broadcast_to(isq, (RB, SW))


def _tc_prep(deg_p, feature):
  return pl.pallas_call(
      _tc_prep_body,
      grid=(NRB,),
      in_specs=[
          pl.BlockSpec((NC, RB, SW), lambda i: (0, i, 0)),
          pl.BlockSpec((RB, D), lambda i: (i, 0)),
      ],
      out_specs=[
          pl.BlockSpec((RB, DE), lambda i: (i, 0)),
          pl.BlockSpec((RB, SW), lambda i: (i, 0)),
      ],
      out_shape=[
          jax.ShapeDtypeStruct((N, DE), jnp.float32),
          jax.ShapeDtypeStruct((N, SW), jnp.float32),
      ],
  )(deg_p, feature)


def _tc_final_body(f_ref, isq16_ref, g_ref, s1_ref, wg_ref, we_ref, o_ref):
  g = g_ref[0] + g_ref[1]
  s1 = s1_ref[0, :, 0:1] + s1_ref[1, :, 0:1]
  f = f_ref[...]
  hf = f * isq16_ref[:, 0:1]
  a = f + hf * s1
  b = hf * g
  dn = (((1,), (1,)), ((), ()))
  r = lax.dot_general(a, wg_ref[...], dn,
                      preferred_element_type=jnp.float32,
                      precision=lax.Precision.HIGHEST)
  r = r + lax.dot_general(b, we_ref[...], dn,
                          preferred_element_type=jnp.float32,
                          precision=lax.Precision.HIGHEST)
  r = jnp.where(r >= 0, r, 0.2 * r)
  nrm = jnp.sqrt(jnp.sum(r * r, axis=1, keepdims=True))
  o_ref[...] = r / jnp.maximum(nrm, 1e-12)


def _tc_final(feature, isq16, g_p, s1_p, W_gcn, W_enh):
  return pl.pallas_call(
      _tc_final_body,
      grid=(NRB,),
      in_specs=[
          pl.BlockSpec((RB, D), lambda i: (i, 0)),
          pl.BlockSpec((RB, SW), lambda i: (i, 0)),
          pl.BlockSpec((NC, RB, D), lambda i: (0, i, 0)),
          pl.BlockSpec((NC, RB, SW), lambda i: (0, i, 0)),
          pl.BlockSpec((D, D), lambda i: (0, 0)),
          pl.BlockSpec((D, D), lambda i: (0, 0)),
      ],
      out_specs=pl.BlockSpec((RB, D), lambda i: (i, 0)),
      out_shape=jax.ShapeDtypeStruct((N, D), jnp.float32),
  )(feature, isq16, g_p, s1_p, W_gcn, W_enh)


def kernel(feature, edge_index, W_gcn, W_enh):
  ei = edge_index.astype(jnp.int32)
  src = ei[0]
  dst = ei[1]
  deg_p = _sc_degree(dst)
  hext, isq16 = _tc_prep(deg_p, feature)
  g_p, s1_p = _sc_edges(src, dst, hext)
  return _tc_final(feature, isq16, g_p, s1_p, W_gcn, W_enh)
